# baseline (speedup 1.0000x reference)
import functools

import jax
import jax.numpy as jnp
from jax import lax
from jax.experimental import pallas as pl
from jax.experimental.pallas import tpu as pltpu
from jax.experimental.pallas import tpu_sc as plsc

_NUM_WORKERS = 32
_CHUNK = 1600


def _emb_body(table_hbm, idx_hbm, out_hbm, idx_v, rows_v, gsems, osems):
    wid = lax.axis_index("s") * 2 + lax.axis_index("c")
    n_per_w = idx_hbm.shape[0] // _NUM_WORKERS
    base = wid * n_per_w
    steps = n_per_w // _CHUNK

    def off(i):
        return base + i * _CHUNK

    def gather(i):
        s = i % 2
        return pltpu.make_async_copy(
            table_hbm.at[idx_v.at[s]], rows_v.at[s], gsems[s])

    def owrite(i):
        s = i % 2
        return pltpu.make_async_copy(
            rows_v.at[s], out_hbm.at[pl.ds(off(i), _CHUNK)], osems[s])

    pltpu.sync_copy(idx_hbm.at[pl.ds(off(0), _CHUNK)], idx_v.at[0])
    gather(0).start()
    pltpu.sync_copy(idx_hbm.at[pl.ds(off(1), _CHUNK)], idx_v.at[1])

    for i in range(steps):
        gather(i).wait()
        owrite(i).start()
        if i + 1 < steps:
            if i >= 1:
                owrite(i - 1).wait()
            gather(i + 1).start()
        if i + 2 < steps:
            pltpu.sync_copy(idx_hbm.at[pl.ds(off(i + 2), _CHUNK)],
                            idx_v.at[i % 2])
    owrite(steps - 2).wait()
    owrite(steps - 1).wait()


def kernel(indices_, tables):
    b, l = indices_.shape
    n = b * l
    d = tables.shape[-1]
    table = tables.reshape(-1, d)
    flat = indices_.reshape(n).astype(jnp.int32)

    mesh = plsc.VectorSubcoreMesh(core_axis_name="c", subcore_axis_name="s")
    run = functools.partial(
        pl.kernel,
        mesh=mesh,
        compiler_params=pltpu.CompilerParams(use_tc_tiling_on_sc=False),
        out_type=jax.ShapeDtypeStruct((n, d), jnp.float32),
        scratch_types=[
            pltpu.VMEM((2, _CHUNK), jnp.int32),
            pltpu.VMEM((2, _CHUNK, d), jnp.float32),
            [pltpu.SemaphoreType.DMA, pltpu.SemaphoreType.DMA],
            [pltpu.SemaphoreType.DMA, pltpu.SemaphoreType.DMA],
        ],
    )(_emb_body)
    out = run(table, flat)
    return out.reshape(b, l, d)


# speedup vs baseline: 1.0061x; 1.0061x over previous
import functools

import jax
import jax.numpy as jnp
from jax import lax
from jax.experimental import pallas as pl
from jax.experimental.pallas import tpu as pltpu
from jax.experimental.pallas import tpu_sc as plsc

_NUM_WORKERS = 32
_CHUNK = 640
_NSLOT = 4


def _emb_body(table_hbm, idx_hbm, out_hbm, idx_v, rows_v, gsems, osems):
    wid = lax.axis_index("s") * 2 + lax.axis_index("c")
    n_per_w = idx_hbm.shape[0] // _NUM_WORKERS
    base = wid * n_per_w
    steps = n_per_w // _CHUNK

    def gather(i):
        s = i % _NSLOT
        return pltpu.make_async_copy(
            table_hbm.at[idx_v.at[pl.ds(i * _CHUNK, _CHUNK)]],
            rows_v.at[s], gsems[s])

    def owrite(i):
        s = i % _NSLOT
        return pltpu.make_async_copy(
            rows_v.at[s], out_hbm.at[pl.ds(base + i * _CHUNK, _CHUNK)],
            osems[s])

    pltpu.sync_copy(idx_hbm.at[pl.ds(base, n_per_w)], idx_v)
    gather(0).start()
    gather(1).start()
    gather(2).start()

    for i in range(steps):
        gather(i).wait()
        owrite(i).start()
        if i + 3 < steps:
            if i >= 1:
                owrite(i - 1).wait()
            gather(i + 3).start()
    for i in range(max(0, steps - 4), steps):
        owrite(i).wait()


def kernel(indices_, tables):
    b, l = indices_.shape
    n = b * l
    d = tables.shape[-1]
    table = tables.reshape(-1, d)
    flat = indices_.reshape(n).astype(jnp.int32)

    n_per_w = n // _NUM_WORKERS
    mesh = plsc.VectorSubcoreMesh(core_axis_name="c", subcore_axis_name="s")
    run = functools.partial(
        pl.kernel,
        mesh=mesh,
        compiler_params=pltpu.CompilerParams(use_tc_tiling_on_sc=False),
        out_type=jax.ShapeDtypeStruct((n, d), jnp.float32),
        scratch_types=[
            pltpu.VMEM((n_per_w,), jnp.int32),
            pltpu.VMEM((_NSLOT, _CHUNK, d), jnp.float32),
            [pltpu.SemaphoreType.DMA] * _NSLOT,
            [pltpu.SemaphoreType.DMA] * _NSLOT,
        ],
    )(_emb_body)
    out = run(table, flat)
    return out.reshape(b, l, d)
